# baseline (device time: 399532 ns/iter reference)
import jax
import jax.numpy as jnp
from jax import lax
from jax.experimental import pallas as pl
from jax.experimental.pallas import tpu as pltpu

N_DEV = 16


def kernel(x, w_mat):
    x = x.astype(jnp.bfloat16)
    w_mat = w_mat.astype(jnp.bfloat16)
    m_per, k = x.shape
    _, n_per = w_mat.shape

    def body(x_ref, w_ref, out_ref, comm_ref, send_sems, recv_sems):
        my_pos = lax.axis_index("i")
        left = lax.rem(my_pos - 1 + N_DEV, N_DEV)
        right = lax.rem(my_pos + 1, N_DEV)

        barrier_sem = pltpu.get_barrier_semaphore()
        for nbr in [left, right]:
            pl.semaphore_signal(
                barrier_sem, inc=1,
                device_id=(nbr,), device_id_type=pl.DeviceIdType.MESH,
            )
        pl.semaphore_wait(barrier_sem, 2)

        comm_ref[0] = x_ref[...]
        out_ref[pl.ds(my_pos * m_per, m_per), :] = jnp.dot(
            comm_ref[0], w_ref[...], preferred_element_type=jnp.float32
        )

        for h in range(N_DEV - 1):
            rdma = pltpu.make_async_remote_copy(
                src_ref=comm_ref.at[h],
                dst_ref=comm_ref.at[h + 1],
                send_sem=send_sems.at[h],
                recv_sem=recv_sems.at[h],
                device_id=(right,),
                device_id_type=pl.DeviceIdType.MESH,
            )
            rdma.start()
            rdma.wait()

            origin = lax.rem(my_pos - (h + 1) + N_DEV, N_DEV)
            out_ref[pl.ds(origin * m_per, m_per), :] = jnp.dot(
                comm_ref[h + 1], w_ref[...], preferred_element_type=jnp.float32
            )

    return pl.pallas_call(
        body,
        out_shape=jax.ShapeDtypeStruct((N_DEV * m_per, n_per), jnp.float32),
        in_specs=[
            pl.BlockSpec(memory_space=pltpu.VMEM),
            pl.BlockSpec(memory_space=pltpu.VMEM),
        ],
        out_specs=pl.BlockSpec(memory_space=pltpu.VMEM),
        scratch_shapes=[
            pltpu.VMEM((N_DEV, m_per, k), jnp.bfloat16),
            pltpu.SemaphoreType.DMA((N_DEV - 1,)),
            pltpu.SemaphoreType.DMA((N_DEV - 1,)),
        ],
        compiler_params=pltpu.CompilerParams(collective_id=0),
    )(x, w_mat)


# device time: 222314 ns/iter; 1.7972x vs baseline; 1.7972x over previous
import jax
import jax.numpy as jnp
from jax import lax
from jax.experimental import pallas as pl
from jax.experimental.pallas import tpu as pltpu

N_DEV = 16
R_HOPS = 8
L_HOPS = 7


def kernel(x, w_mat):
    x = x.astype(jnp.bfloat16)
    w_mat = w_mat.astype(jnp.bfloat16)
    m_per, k = x.shape
    _, n_per = w_mat.shape

    def body(x_ref, w_ref, out_ref, comm_ref, ss_r, rs_r, ss_l, rs_l):
        my_pos = lax.axis_index("i")
        left = lax.rem(my_pos - 1 + N_DEV, N_DEV)
        right = lax.rem(my_pos + 1, N_DEV)

        barrier_sem = pltpu.get_barrier_semaphore()
        for nbr in [left, right]:
            pl.semaphore_signal(
                barrier_sem, inc=1,
                device_id=(nbr,), device_id_type=pl.DeviceIdType.MESH,
            )
        pl.semaphore_wait(barrier_sem, 2)

        comm_ref[0] = x_ref[...]

        def gemm(slot, origin):
            out_ref[pl.ds(origin * m_per, m_per), :] = jnp.dot(
                comm_ref[slot], w_ref[...], preferred_element_type=jnp.float32
            )

        sends = []
        for h in range(1, R_HOPS + 1):
            rd_r = pltpu.make_async_remote_copy(
                src_ref=comm_ref.at[h - 1],
                dst_ref=comm_ref.at[h],
                send_sem=ss_r.at[h - 1],
                recv_sem=rs_r.at[h - 1],
                device_id=(right,),
                device_id_type=pl.DeviceIdType.MESH,
            )
            rd_r.start()
            sends.append(rd_r)
            rd_l = None
            if h <= L_HOPS:
                src_slot = 0 if h == 1 else R_HOPS + h - 1
                rd_l = pltpu.make_async_remote_copy(
                    src_ref=comm_ref.at[src_slot],
                    dst_ref=comm_ref.at[R_HOPS + h],
                    send_sem=ss_l.at[h - 1],
                    recv_sem=rs_l.at[h - 1],
                    device_id=(left,),
                    device_id_type=pl.DeviceIdType.MESH,
                )
                rd_l.start()
                sends.append(rd_l)

            if h == 1:
                gemm(0, my_pos)
            else:
                gemm(h - 1, lax.rem(my_pos - (h - 1) + N_DEV, N_DEV))
                gemm(R_HOPS + h - 1, lax.rem(my_pos + (h - 1), N_DEV))

            rd_r.wait_recv()
            if rd_l is not None:
                rd_l.wait_recv()

        gemm(R_HOPS, lax.rem(my_pos - R_HOPS + N_DEV, N_DEV))
        gemm(R_HOPS + L_HOPS, lax.rem(my_pos + L_HOPS, N_DEV))

        for rd in sends:
            rd.wait_send()

    return pl.pallas_call(
        body,
        out_shape=jax.ShapeDtypeStruct((N_DEV * m_per, n_per), jnp.float32),
        in_specs=[
            pl.BlockSpec(memory_space=pltpu.VMEM),
            pl.BlockSpec(memory_space=pltpu.VMEM),
        ],
        out_specs=pl.BlockSpec(memory_space=pltpu.VMEM),
        scratch_shapes=[
            pltpu.VMEM((N_DEV, m_per, k), jnp.bfloat16),
            pltpu.SemaphoreType.DMA((R_HOPS,)),
            pltpu.SemaphoreType.DMA((R_HOPS,)),
            pltpu.SemaphoreType.DMA((L_HOPS,)),
            pltpu.SemaphoreType.DMA((L_HOPS,)),
        ],
        compiler_params=pltpu.CompilerParams(collective_id=0),
    )(x, w_mat)


# device time: 188664 ns/iter; 2.1177x vs baseline; 1.1784x over previous
import jax
import jax.numpy as jnp
from jax import lax
from jax.experimental import pallas as pl
from jax.experimental.pallas import tpu as pltpu

N_DEV = 16
R_HOPS = 8
L_HOPS = 8


def kernel(x, w_mat):
    x = x.astype(jnp.bfloat16)
    w_mat = w_mat.astype(jnp.bfloat16)
    m_per, k = x.shape
    _, n_per = w_mat.shape
    mh = m_per // 2

    def body(x_ref, w_ref, out_ref, comm_ref,
             ss_ra, rs_ra, ss_rb, rs_rb, ss_la, rs_la, ss_lb, rs_lb):
        my_pos = lax.axis_index("i")
        left = lax.rem(my_pos - 1 + N_DEV, N_DEV)
        right = lax.rem(my_pos + 1, N_DEV)

        A = pl.ds(0, mh)
        B = pl.ds(mh, mh)

        barrier_sem = pltpu.get_barrier_semaphore()
        for nbr in [left, right]:
            pl.semaphore_signal(
                barrier_sem, inc=1,
                device_id=(nbr,), device_id_type=pl.DeviceIdType.MESH,
            )
        comm_ref[0] = x_ref[...]
        pl.semaphore_wait(barrier_sem, 2)

        def mk(src_slot, dst_slot, rows, ss, rs, idx, target):
            return pltpu.make_async_remote_copy(
                src_ref=comm_ref.at[src_slot, rows],
                dst_ref=comm_ref.at[dst_slot, rows],
                send_sem=ss.at[idx],
                recv_sem=rs.at[idx],
                device_id=(target,),
                device_id_type=pl.DeviceIdType.MESH,
            )

        def gemm(slot, origin):
            out_ref[pl.ds(origin * m_per, m_per), :] = jnp.dot(
                comm_ref[slot], w_ref[...],
                preferred_element_type=jnp.float32,
            )

        rd_ra = [None] * (R_HOPS + 1)
        rd_rb = [None] * R_HOPS
        rd_la = [None] * L_HOPS
        rd_lb = [None] * (L_HOPS + 1)

        for h in range(1, R_HOPS + 1):
            if h > 1:
                rd_ra[h - 1].wait_recv()
            rd_ra[h] = mk(h - 1, h, A, ss_ra, rs_ra, h - 1, right)
            rd_ra[h].start()

            if h > 1:
                rd_rb[h - 1].wait_recv()
            if h <= 7:
                rd_rb[h] = mk(h - 1, h, B, ss_rb, rs_rb, h - 1, right)
                rd_rb[h].start()

            if h > 1:
                rd_la[h - 1].wait_recv()
            if h <= 7:
                src = 0 if h == 1 else R_HOPS + h - 1
                rd_la[h] = mk(src, R_HOPS + h, A, ss_la, rs_la, h - 1, left)
                rd_la[h].start()

            if h > 1:
                rd_lb[h - 1].wait_recv()
            src = 0 if h == 1 else R_HOPS + h - 1
            dst = R_HOPS + h if h <= 7 else R_HOPS
            if h == 8:
                src = R_HOPS + 7
            rd_lb[h] = mk(src, dst, B, ss_lb, rs_lb, h - 1, left)
            rd_lb[h].start()

            if h == 1:
                gemm(0, my_pos)
            else:
                gemm(h - 1, lax.rem(my_pos - (h - 1) + N_DEV, N_DEV))
                gemm(R_HOPS + h - 1, lax.rem(my_pos + (h - 1), N_DEV))

        rd_ra[R_HOPS].wait_recv()
        rd_lb[L_HOPS].wait_recv()
        gemm(R_HOPS, lax.rem(my_pos - R_HOPS + N_DEV, N_DEV))

        for rd in rd_ra[1:] + rd_rb[1:] + rd_la[1:] + rd_lb[1:]:
            rd.wait_send()

    return pl.pallas_call(
        body,
        out_shape=jax.ShapeDtypeStruct((N_DEV * m_per, n_per), jnp.float32),
        in_specs=[
            pl.BlockSpec(memory_space=pltpu.VMEM),
            pl.BlockSpec(memory_space=pltpu.VMEM),
        ],
        out_specs=pl.BlockSpec(memory_space=pltpu.VMEM),
        scratch_shapes=[
            pltpu.VMEM((N_DEV, m_per, k), jnp.bfloat16),
            pltpu.SemaphoreType.DMA((R_HOPS,)),
            pltpu.SemaphoreType.DMA((R_HOPS,)),
            pltpu.SemaphoreType.DMA((7,)),
            pltpu.SemaphoreType.DMA((7,)),
            pltpu.SemaphoreType.DMA((7,)),
            pltpu.SemaphoreType.DMA((7,)),
            pltpu.SemaphoreType.DMA((L_HOPS,)),
            pltpu.SemaphoreType.DMA((L_HOPS,)),
        ],
        compiler_params=pltpu.CompilerParams(collective_id=0),
    )(x, w_mat)
